# Initial kernel scaffold; baseline (speedup 1.0000x reference)
import jax
import jax.numpy as jnp
from jax.experimental import pallas as pl


def kernel(desc1, desc2):
    raise NotImplementedError("write your pallas kernel here")


# speedup vs baseline: 1.4716x; 1.4716x over previous
import jax
import jax.numpy as jnp
from jax.experimental import pallas as pl
from jax.experimental.pallas import tpu as pltpu

_QB = 1024
_KB = 2048
_NK = 5


def _nn_body(d1_ref, d2t_ref, dist_ref, idx_ref, min_ref, arg_ref, *, n_keys):
    k = pl.program_id(1)

    @pl.when(k == 0)
    def _init():
        min_ref[...] = jnp.full((_QB, 1), jnp.inf, jnp.float32)
        arg_ref[...] = jnp.zeros((_QB, 1), jnp.int32)

    d1 = d1_ref[...]
    d2t = d2t_ref[...]
    d2sq = jnp.sum(d2t * d2t, axis=0)
    dot = jnp.dot(d1, d2t, preferred_element_type=jnp.float32)
    lane = jax.lax.broadcasted_iota(jnp.int32, (_QB, _KB), 1)
    gidx = lane + k * _KB
    scores = jnp.where(gidx < n_keys, d2sq[None, :] - 2.0 * dot, jnp.inf)

    cmin = jnp.min(scores, axis=1, keepdims=True)
    carg = jnp.min(jnp.where(scores == cmin, lane, jnp.int32(2**30)),
                   axis=1, keepdims=True) + k * _KB

    better = cmin < min_ref[...]
    arg_ref[...] = jnp.where(better, carg, arg_ref[...])
    min_ref[...] = jnp.where(better, cmin, min_ref[...])

    @pl.when(k == _NK - 1)
    def _finish():
        q_sq = jnp.sum(d1 * d1, axis=1, keepdims=True)
        dist_ref[...] = jnp.sqrt(jnp.maximum(min_ref[...] + q_sq, 0.0))
        idx_ref[...] = arg_ref[...]


def kernel(desc1, desc2):
    import functools
    q, d = desc1.shape
    n_keys = desc2.shape[0]
    q_pad = ((q + _QB - 1) // _QB) * _QB
    k_pad = _NK * _KB
    d1p = jnp.pad(desc1, ((0, q_pad - q), (0, 0)))
    d2t = jnp.pad(desc2, ((0, k_pad - n_keys), (0, 0))).T

    dists, idxs = pl.pallas_call(
        functools.partial(_nn_body, n_keys=n_keys),
        grid=(q_pad // _QB, _NK),
        in_specs=[
            pl.BlockSpec((_QB, d), lambda i, j: (i, 0)),
            pl.BlockSpec((d, _KB), lambda i, j: (0, j)),
        ],
        out_specs=[
            pl.BlockSpec((_QB, 1), lambda i, j: (i, 0)),
            pl.BlockSpec((_QB, 1), lambda i, j: (i, 0)),
        ],
        out_shape=[
            jax.ShapeDtypeStruct((q_pad, 1), jnp.float32),
            jax.ShapeDtypeStruct((q_pad, 1), jnp.int32),
        ],
        scratch_shapes=[
            pltpu.VMEM((_QB, 1), jnp.float32),
            pltpu.VMEM((_QB, 1), jnp.int32),
        ],
    )(d1p, d2t)

    idxs_in_1 = jnp.arange(q, dtype=jnp.int32)
    matches_idxs = jnp.stack([idxs_in_1, idxs[:q].reshape(-1)], axis=1)
    return dists[:q], matches_idxs


# speedup vs baseline: 1.5557x; 1.0572x over previous
import functools

import jax
import jax.numpy as jnp
from jax.experimental import pallas as pl
from jax.experimental.pallas import tpu as pltpu

_QB = 1024
_KB = 2048
_NK = 5


def _nn_body(d1_ref, d2tm_ref, dist_ref, idx_ref, *, n_keys):
    d1 = d1_ref[...]

    best_val = jnp.full((_QB, 1), jnp.inf, jnp.float32)
    best_idx = jnp.zeros((_QB, 1), jnp.int32)
    lane = jax.lax.broadcasted_iota(jnp.int32, (_QB, _KB), 1)

    for j in range(_NK):
        d2tm = d2tm_ref[:, j * _KB:(j + 1) * _KB]
        d2sq = 0.25 * jnp.sum(d2tm * d2tm, axis=0)
        kvec = jax.lax.broadcasted_iota(jnp.int32, (1, _KB), 1) + j * _KB
        d2sq = d2sq + jnp.where(kvec[0] < n_keys, 0.0, jnp.inf)
        dot = jnp.dot(d1, d2tm, preferred_element_type=jnp.float32)
        scores = dot + d2sq[None, :]

        cmin = jnp.min(scores, axis=1, keepdims=True)
        carg = jnp.min(jnp.where(scores == cmin, lane, jnp.int32(2**30)),
                       axis=1, keepdims=True) + j * _KB

        better = cmin < best_val
        best_idx = jnp.where(better, carg, best_idx)
        best_val = jnp.where(better, cmin, best_val)

    q_sq = jnp.sum(d1 * d1, axis=1, keepdims=True)
    dist_ref[...] = jnp.sqrt(jnp.maximum(best_val + q_sq, 0.0))
    idx_ref[...] = best_idx


def kernel(desc1, desc2):
    q, d = desc1.shape
    n_keys = desc2.shape[0]
    q_pad = ((q + _QB - 1) // _QB) * _QB
    k_pad = _NK * _KB
    d1p = jnp.pad(desc1, ((0, q_pad - q), (0, 0)))
    d2tm = (-2.0 * jnp.pad(desc2, ((0, k_pad - n_keys), (0, 0)))).T

    dists, idxs = pl.pallas_call(
        functools.partial(_nn_body, n_keys=n_keys),
        grid=(q_pad // _QB,),
        in_specs=[
            pl.BlockSpec((_QB, d), lambda i: (i, 0)),
            pl.BlockSpec((d, k_pad), lambda i: (0, 0)),
        ],
        out_specs=[
            pl.BlockSpec((_QB, 1), lambda i: (i, 0)),
            pl.BlockSpec((_QB, 1), lambda i: (i, 0)),
        ],
        out_shape=[
            jax.ShapeDtypeStruct((q_pad, 1), jnp.float32),
            jax.ShapeDtypeStruct((q_pad, 1), jnp.int32),
        ],
    )(d1p, d2tm)

    idxs_in_1 = jnp.arange(q, dtype=jnp.int32)
    matches_idxs = jnp.stack([idxs_in_1, idxs[:q].reshape(-1)], axis=1)
    return dists[:q], matches_idxs


# speedup vs baseline: 1.9231x; 1.2361x over previous
import functools

import jax
import jax.numpy as jnp
from jax.experimental import pallas as pl
from jax.experimental.pallas import tpu as pltpu

_QB = 2000
_KB = 2048
_NK = 5


def _nn_body(d1_ref, d2tm_ref, dist_ref, idx_ref, *, n_keys):
    d1 = d1_ref[...]

    best_val = jnp.full((_QB, 1), jnp.inf, jnp.float32)
    best_idx = jnp.zeros((_QB, 1), jnp.float32)
    lane = jax.lax.broadcasted_iota(
        jnp.int32, (_QB, _KB), 1).astype(jnp.float32)

    for j in range(_NK):
        d2tm = d2tm_ref[:, j * _KB:(j + 1) * _KB]
        d2sq = 0.25 * jnp.sum(d2tm * d2tm, axis=0)
        kvec = jax.lax.broadcasted_iota(jnp.int32, (1, _KB), 1) + j * _KB
        d2sq = d2sq + jnp.where(kvec[0] < n_keys, 0.0, jnp.inf)
        dot = jnp.dot(d1, d2tm, preferred_element_type=jnp.float32)
        scores = dot + d2sq[None, :]

        cmin = jnp.min(scores, axis=1, keepdims=True)
        carg = jnp.min(jnp.where(scores == cmin, lane, jnp.float32(2**30)),
                       axis=1, keepdims=True) + jnp.float32(j * _KB)

        better = cmin < best_val
        best_idx = jnp.where(better, carg, best_idx)
        best_val = jnp.where(better, cmin, best_val)

    q_sq = jnp.sum(d1 * d1, axis=1, keepdims=True)
    dist_ref[...] = jnp.sqrt(jnp.maximum(best_val + q_sq, 0.0))
    idx_ref[...] = best_idx.astype(jnp.int32)


def kernel(desc1, desc2):
    q, d = desc1.shape
    n_keys = desc2.shape[0]
    q_pad = ((q + _QB - 1) // _QB) * _QB
    k_pad = _NK * _KB
    d1p = jnp.pad(desc1, ((0, q_pad - q), (0, 0)))
    d2tm = (-2.0 * jnp.pad(desc2, ((0, k_pad - n_keys), (0, 0)))).T

    dists, idxs = pl.pallas_call(
        functools.partial(_nn_body, n_keys=n_keys),
        grid=(q_pad // _QB,),
        in_specs=[
            pl.BlockSpec((_QB, d), lambda i: (i, 0)),
            pl.BlockSpec((d, k_pad), lambda i: (0, 0)),
        ],
        out_specs=[
            pl.BlockSpec((_QB, 1), lambda i: (i, 0)),
            pl.BlockSpec((_QB, 1), lambda i: (i, 0)),
        ],
        out_shape=[
            jax.ShapeDtypeStruct((q_pad, 1), jnp.float32),
            jax.ShapeDtypeStruct((q_pad, 1), jnp.int32),
        ],
    )(d1p, d2tm)

    idxs_in_1 = jnp.arange(q, dtype=jnp.int32)
    matches_idxs = jnp.stack([idxs_in_1, idxs[:q].reshape(-1)], axis=1)
    return dists[:q], matches_idxs
